# initial kernel scaffold (unmeasured)
import functools

import jax
import jax.numpy as jnp
from jax import lax
from jax.experimental import pallas as pl
from jax.experimental.pallas import tpu as pltpu

N_DEV = 4
B, SQ, D = 2, 512, 768
HL, DH = 8, 64
SKV_LOC = 512
WINDOW, NGLOB = 128, 32


def kernel(x, Wq, K_ext, V_ext, Wo):
    xb = x.astype(jnp.bfloat16)
    wqb = Wq.astype(jnp.bfloat16)
    kb = K_ext.astype(jnp.bfloat16)
    vb = V_ext.astype(jnp.bfloat16)
    wob = Wo.astype(jnp.bfloat16)

    def body(x_ref, wq_ref, k_ref, v_ref, wo_ref, out_ref,
             kg_ref, vg_ref, red_ref, pbf_ref,
             ksend_sems, vsend_sems, kg_recv_sems, vg_recv_sems,
             rsend_sems, rrecv_sems):
        my = lax.axis_index("i")

        barrier_sem = pltpu.get_barrier_semaphore()
        for d in range(1, N_DEV):
            pl.semaphore_signal(
                barrier_sem, inc=1,
                device_id=((my + d) % N_DEV,),
                device_id_type=pl.DeviceIdType.MESH,
            )
        pl.semaphore_wait(barrier_sem, N_DEV - 1)

        sends = []
        for d in range(1, N_DEV):
            peer = (my + d) % N_DEV
            krd = pltpu.make_async_remote_copy(
                src_ref=k_ref.at[:, :, pl.ds(peer * HL, HL), :],
                dst_ref=kg_ref.at[my],
                send_sem=ksend_sems.at[d - 1],
                recv_sem=kg_recv_sems.at[my],
                device_id=(peer,),
                device_id_type=pl.DeviceIdType.MESH,
            )
            krd.start()
            vrd = pltpu.make_async_remote_copy(
                src_ref=v_ref.at[:, :, pl.ds(peer * HL, HL), :],
                dst_ref=vg_ref.at[my],
                send_sem=vsend_sems.at[d - 1],
                recv_sem=vg_recv_sems.at[my],
                device_id=(peer,),
                device_id_type=pl.DeviceIdType.MESH,
            )
            vrd.start()
            sends += [krd, vrd]

        kg_ref[pl.ds(my, 1)] = k_ref[:, :, pl.ds(my * HL, HL), :][None]
        vg_ref[pl.ds(my, 1)] = v_ref[:, :, pl.ds(my * HL, HL), :][None]

        q = lax.dot_general(
            x_ref[...].reshape(B * SQ, D), wq_ref[...],
            (((1,), (0,)), ((), ())), preferred_element_type=jnp.float32,
        )
        q = (q * 0.125).astype(jnp.bfloat16).reshape(B, SQ, HL, DH)

        for j in range(N_DEV):
            @pl.when(my != j)
            def _(j=j):
                for buf, sem in ((kg_ref, kg_recv_sems), (vg_ref, vg_recv_sems)):
                    rd = pltpu.make_async_remote_copy(
                        src_ref=buf.at[j], dst_ref=buf.at[j],
                        send_sem=sem.at[j], recv_sem=sem.at[j],
                        device_id=(j,), device_id_type=pl.DeviceIdType.MESH,
                    )
                    rd.wait_recv()

        qi = lax.broadcasted_iota(jnp.int32, (SQ, SKV_LOC), 0)
        kix = lax.broadcasted_iota(jnp.int32, (SQ, SKV_LOC), 1)
        parts = []
        for b in range(B):
            ctx_h = []
            for h in range(HL):
                qbh = q[b, :, h, :]
                m = jnp.full((SQ, 1), -1e30, jnp.float32)
                l = jnp.zeros((SQ, 1), jnp.float32)
                acc = jnp.zeros((SQ, DH), jnp.float32)
                for j in range(N_DEV):
                    ki = kix + j * SKV_LOC
                    mask = (jnp.abs(qi - ki) <= WINDOW) | (ki < NGLOB) | (qi < NGLOB)
                    kj = kg_ref[j, b, :, h, :]
                    vj = vg_ref[j, b, :, h, :]
                    s = lax.dot_general(
                        qbh, kj, (((1,), (1,)), ((), ())),
                        preferred_element_type=jnp.float32,
                    )
                    s = jnp.where(mask, s, -1e9)
                    m_new = jnp.maximum(m, s.max(axis=1, keepdims=True))
                    alpha = jnp.exp(m - m_new)
                    p = jnp.exp(s - m_new)
                    l = l * alpha + p.sum(axis=1, keepdims=True)
                    acc = acc * alpha + lax.dot_general(
                        p.astype(jnp.bfloat16), vj, (((1,), (0,)), ((), ())),
                        preferred_element_type=jnp.float32,
                    )
                    m = m_new
                ctx_h.append(acc / l)
            ctx_b = jnp.concatenate(ctx_h, axis=1).astype(jnp.bfloat16)
            part_b = lax.dot_general(
                ctx_b, wo_ref[...], (((1,), (0,)), ((), ())),
                preferred_element_type=jnp.float32,
            )
            parts.append(part_b)
            out_ref[b] = part_b
            pbf_ref[b] = part_b.astype(jnp.bfloat16)

        for d in range(1, N_DEV):
            peer = (my + d) % N_DEV
            rrd = pltpu.make_async_remote_copy(
                src_ref=pbf_ref,
                dst_ref=red_ref.at[my],
                send_sem=rsend_sems.at[d - 1],
                recv_sem=rrecv_sems.at[my],
                device_id=(peer,),
                device_id_type=pl.DeviceIdType.MESH,
            )
            rrd.start()
            sends.append(rrd)

        for j in range(N_DEV):
            @pl.when(my != j)
            def _(j=j):
                rd = pltpu.make_async_remote_copy(
                    src_ref=red_ref.at[j], dst_ref=red_ref.at[j],
                    send_sem=rrecv_sems.at[j], recv_sem=rrecv_sems.at[j],
                    device_id=(j,), device_id_type=pl.DeviceIdType.MESH,
                )
                rd.wait_recv()
                out_ref[...] = out_ref[...] + red_ref[j].astype(jnp.float32)

        for rd in sends:
            rd.wait_send()

    return pl.pallas_call(
        body,
        out_shape=jax.ShapeDtypeStruct((B, SQ, D), jnp.float32),
        in_specs=[pl.BlockSpec(memory_space=pltpu.VMEM)] * 5,
        out_specs=pl.BlockSpec(memory_space=pltpu.VMEM),
        scratch_shapes=[
            pltpu.VMEM((N_DEV, B, SKV_LOC, HL, DH), jnp.bfloat16),
            pltpu.VMEM((N_DEV, B, SKV_LOC, HL, DH), jnp.bfloat16),
            pltpu.VMEM((N_DEV, B, SQ, D), jnp.bfloat16),
            pltpu.VMEM((B, SQ, D), jnp.bfloat16),
            pltpu.SemaphoreType.DMA((N_DEV - 1,)),
            pltpu.SemaphoreType.DMA((N_DEV - 1,)),
            pltpu.SemaphoreType.DMA((N_DEV,)),
            pltpu.SemaphoreType.DMA((N_DEV,)),
            pltpu.SemaphoreType.DMA((N_DEV - 1,)),
            pltpu.SemaphoreType.DMA((N_DEV,)),
        ],
        compiler_params=pltpu.CompilerParams(collective_id=0),
    )(xb, wqb, kb, vb, wob)


# baseline (device time: 135197 ns/iter reference)
import jax
import jax.numpy as jnp
from jax import lax
from jax.experimental import pallas as pl
from jax.experimental.pallas import tpu as pltpu

N_DEV = 4
B, SQ, D = 2, 512, 768
HL, DH = 8, 64
SKV_LOC = 512
WINDOW, NGLOB = 128, 32


def kernel(x, Wq, K_ext, V_ext, Wo):
    xt = x.astype(jnp.bfloat16).transpose(0, 2, 1)
    wqb = Wq.astype(jnp.bfloat16)
    kt = K_ext.astype(jnp.bfloat16).transpose(0, 2, 3, 1)
    vt = V_ext.astype(jnp.bfloat16).transpose(0, 2, 3, 1)
    wob = Wo.astype(jnp.bfloat16)

    def body(xt_ref, wq_ref, kt_ref, vt_ref, wo_ref, out_ref,
             kg_ref, vg_ref, qt_ref, red_ref, pbf_ref,
             ksend_sems, vsend_sems, kg_recv_sems, vg_recv_sems,
             rsend_sems, rrecv_sems):
        my = lax.axis_index("i")

        barrier_sem = pltpu.get_barrier_semaphore()
        for d in range(1, N_DEV):
            pl.semaphore_signal(
                barrier_sem, inc=1,
                device_id=((my + d) % N_DEV,),
                device_id_type=pl.DeviceIdType.MESH,
            )
        pl.semaphore_wait(barrier_sem, N_DEV - 1)

        sends = []
        for d in range(1, N_DEV):
            peer = (my + d) % N_DEV
            for src, dst, ssem, rsem in (
                (kt_ref, kg_ref, ksend_sems, kg_recv_sems),
                (vt_ref, vg_ref, vsend_sems, vg_recv_sems),
            ):
                rd = pltpu.make_async_remote_copy(
                    src_ref=src.at[:, pl.ds(peer * HL, HL)],
                    dst_ref=dst.at[my],
                    send_sem=ssem.at[d - 1],
                    recv_sem=rsem.at[my],
                    device_id=(peer,),
                    device_id_type=pl.DeviceIdType.MESH,
                )
                rd.start()
                sends.append(rd)

        kg_ref[pl.ds(my, 1)] = kt_ref[:, pl.ds(my * HL, HL)][None]
        vg_ref[pl.ds(my, 1)] = vt_ref[:, pl.ds(my * HL, HL)][None]

        for b in range(B):
            qt_b = lax.dot_general(
                wq_ref[...], xt_ref[b], (((0,), (0,)), ((), ())),
                preferred_element_type=jnp.float32,
            )
            qt_ref[b] = (qt_b * 0.125).astype(jnp.bfloat16)

        out_ref[...] = jnp.zeros((B, SQ, D), jnp.float32)

        for j in range(N_DEV):
            @pl.when(my != j)
            def _(j=j):
                for buf, sem in ((kg_ref, kg_recv_sems), (vg_ref, vg_recv_sems)):
                    rd = pltpu.make_async_remote_copy(
                        src_ref=buf.at[j], dst_ref=buf.at[j],
                        send_sem=sem.at[j], recv_sem=sem.at[j],
                        device_id=(j,), device_id_type=pl.DeviceIdType.MESH,
                    )
                    rd.wait_recv()

        kvi = lax.broadcasted_iota(jnp.int32, (SKV_LOC, SQ), 0)
        qidx = lax.broadcasted_iota(jnp.int32, (SKV_LOC, SQ), 1)
        for b in range(B):
            for h in range(HL):
                qt_h = qt_ref[b, pl.ds(h * DH, DH), :]
                m = jnp.full((1, SQ), -1e30, jnp.float32)
                l = jnp.zeros((1, SQ), jnp.float32)
                accT = jnp.zeros((DH, SQ), jnp.float32)
                for j in range(N_DEV):
                    ki = kvi + j * SKV_LOC
                    maskT = (jnp.abs(qidx - ki) <= WINDOW) | (ki < NGLOB) | (qidx < NGLOB)
                    sT = lax.dot_general(
                        kg_ref[j, b, h], qt_h, (((0,), (0,)), ((), ())),
                        preferred_element_type=jnp.float32,
                    )
                    sT = jnp.where(maskT, sT, -1e9)
                    m_new = jnp.maximum(m, sT.max(axis=0, keepdims=True))
                    alpha = jnp.exp(m - m_new)
                    p = jnp.exp(sT - m_new)
                    l = l * alpha + p.sum(axis=0, keepdims=True)
                    accT = accT * alpha + lax.dot_general(
                        vg_ref[j, b, h], p.astype(jnp.bfloat16),
                        (((1,), (0,)), ((), ())),
                        preferred_element_type=jnp.float32,
                    )
                    m = m_new
                ctxT_h = (accT / l).astype(jnp.bfloat16)
                out_ref[b] = out_ref[b] + lax.dot_general(
                    ctxT_h, wo_ref[pl.ds(h * DH, DH), :],
                    (((0,), (0,)), ((), ())),
                    preferred_element_type=jnp.float32,
                )

        pbf_ref[...] = out_ref[...].astype(jnp.bfloat16)

        for d in range(1, N_DEV):
            peer = (my + d) % N_DEV
            rd = pltpu.make_async_remote_copy(
                src_ref=pbf_ref,
                dst_ref=red_ref.at[my],
                send_sem=rsend_sems.at[d - 1],
                recv_sem=rrecv_sems.at[my],
                device_id=(peer,),
                device_id_type=pl.DeviceIdType.MESH,
            )
            rd.start()
            sends.append(rd)

        for j in range(N_DEV):
            @pl.when(my != j)
            def _(j=j):
                rd = pltpu.make_async_remote_copy(
                    src_ref=red_ref.at[j], dst_ref=red_ref.at[j],
                    send_sem=rrecv_sems.at[j], recv_sem=rrecv_sems.at[j],
                    device_id=(j,), device_id_type=pl.DeviceIdType.MESH,
                )
                rd.wait_recv()
                out_ref[...] = out_ref[...] + red_ref[j].astype(jnp.float32)

        for rd in sends:
            rd.wait_send()

    return pl.pallas_call(
        body,
        out_shape=jax.ShapeDtypeStruct((B, SQ, D), jnp.float32),
        in_specs=[pl.BlockSpec(memory_space=pltpu.VMEM)] * 5,
        out_specs=pl.BlockSpec(memory_space=pltpu.VMEM),
        scratch_shapes=[
            pltpu.VMEM((N_DEV, B, HL, DH, SKV_LOC), jnp.bfloat16),
            pltpu.VMEM((N_DEV, B, HL, DH, SKV_LOC), jnp.bfloat16),
            pltpu.VMEM((B, HL * DH, SQ), jnp.bfloat16),
            pltpu.VMEM((N_DEV, B, SQ, D), jnp.bfloat16),
            pltpu.VMEM((B, SQ, D), jnp.bfloat16),
            pltpu.SemaphoreType.DMA((N_DEV - 1,)),
            pltpu.SemaphoreType.DMA((N_DEV - 1,)),
            pltpu.SemaphoreType.DMA((N_DEV,)),
            pltpu.SemaphoreType.DMA((N_DEV,)),
            pltpu.SemaphoreType.DMA((N_DEV - 1,)),
            pltpu.SemaphoreType.DMA((N_DEV,)),
        ],
        compiler_params=pltpu.CompilerParams(collective_id=0),
    )(xt, wqb, kt, vt, wob)


# device time: 104451 ns/iter; 1.2944x vs baseline; 1.2944x over previous
import jax
import jax.numpy as jnp
from jax import lax
from jax.experimental import pallas as pl
from jax.experimental.pallas import tpu as pltpu

N_DEV = 4
B, SQ, D = 2, 512, 768
HL, DH = 8, 64
SKV_LOC = 512
WINDOW, NGLOB = 128, 32
QROWS = SQ // N_DEV


def kernel(x, Wq, K_ext, V_ext, Wo):
    xt = x.astype(jnp.bfloat16).transpose(0, 2, 1)
    wqb = Wq.astype(jnp.bfloat16)
    kt = K_ext.astype(jnp.bfloat16).transpose(0, 2, 3, 1)
    vt = V_ext.astype(jnp.bfloat16).transpose(0, 2, 3, 1)
    wob = Wo.astype(jnp.bfloat16)

    def body(xt_ref, wq_ref, kt_ref, vt_ref, wo_ref, out_ref,
             kg_ref, vg_ref, qt_ref, macc_ref, m_ref, l_ref,
             rs_ref, ag_ref, pbf_ref, agb_ref,
             ksend_sems, vsend_sems, kg_recv_sems, vg_recv_sems,
             rs_send_sems, rs_recv_sems, ag_send_sems, ag_recv_sems):
        my = lax.axis_index("i")

        barrier_sem = pltpu.get_barrier_semaphore()
        for d in range(1, N_DEV):
            pl.semaphore_signal(
                barrier_sem, inc=1,
                device_id=((my + d) % N_DEV,),
                device_id_type=pl.DeviceIdType.MESH,
            )
        pl.semaphore_wait(barrier_sem, N_DEV - 1)

        sends = []
        for d in range(1, N_DEV):
            peer = (my + d) % N_DEV
            for src, dst, ssem, rsem in (
                (kt_ref, kg_ref, ksend_sems, kg_recv_sems),
                (vt_ref, vg_ref, vsend_sems, vg_recv_sems),
            ):
                rd = pltpu.make_async_remote_copy(
                    src_ref=src.at[:, pl.ds(peer * HL, HL)],
                    dst_ref=dst.at[my],
                    send_sem=ssem.at[d - 1],
                    recv_sem=rsem.at[my],
                    device_id=(peer,),
                    device_id_type=pl.DeviceIdType.MESH,
                )
                rd.start()
                sends.append(rd)

        kg_ref[pl.ds(my, 1)] = kt_ref[:, pl.ds(my * HL, HL)][None]
        vg_ref[pl.ds(my, 1)] = vt_ref[:, pl.ds(my * HL, HL)][None]

        rs_ref[pl.ds(my, 1)] = jnp.zeros((1, B, QROWS, D), jnp.bfloat16)

        for b in range(B):
            qt_b = lax.dot_general(
                wq_ref[...], xt_ref[b], (((0,), (0,)), ((), ())),
                preferred_element_type=jnp.float32,
            )
            qt_ref[b] = (qt_b * 0.125).astype(jnp.bfloat16)

        out_ref[...] = jnp.zeros((B, SQ, D), jnp.float32)
        macc_ref[...] = jnp.zeros((B, HL, DH, SQ), jnp.float32)
        m_ref[...] = jnp.full((B * HL, SQ), -1e30, jnp.float32)
        l_ref[...] = jnp.zeros((B * HL, SQ), jnp.float32)

        kvi = lax.broadcasted_iota(jnp.int32, (SKV_LOC, SQ), 0)
        qidx = lax.broadcasted_iota(jnp.int32, (SKV_LOC, SQ), 1)
        for d in (0, 3, 2, 1):
            j = (my + d) % N_DEV

            if d != 0:
                for buf, sem in ((kg_ref, kg_recv_sems), (vg_ref, vg_recv_sems)):
                    rd = pltpu.make_async_remote_copy(
                        src_ref=buf.at[j], dst_ref=buf.at[j],
                        send_sem=sem.at[j], recv_sem=sem.at[j],
                        device_id=(j,), device_id_type=pl.DeviceIdType.MESH,
                    )
                    rd.wait_recv()

            ki = kvi + j * SKV_LOC
            maskT = (jnp.abs(qidx - ki) <= WINDOW) | (ki < NGLOB) | (qidx < NGLOB)
            for b in range(B):
                for h in range(HL):
                    qt_h = qt_ref[b, pl.ds(h * DH, DH), :]
                    sT = lax.dot_general(
                        kg_ref[j, b, h], qt_h, (((0,), (0,)), ((), ())),
                        preferred_element_type=jnp.float32,
                    )
                    sT = jnp.where(maskT, sT, -1e9)
                    m = m_ref[pl.ds(b * HL + h, 1), :]
                    m_new = jnp.maximum(m, sT.max(axis=0, keepdims=True))
                    alpha = jnp.exp(m - m_new)
                    p = jnp.exp(sT - m_new)
                    l_ref[pl.ds(b * HL + h, 1), :] = (
                        l_ref[pl.ds(b * HL + h, 1), :] * alpha
                        + p.sum(axis=0, keepdims=True)
                    )
                    macc_ref[b, h] = macc_ref[b, h] * alpha + lax.dot_general(
                        vg_ref[j, b, h], p.astype(jnp.bfloat16),
                        (((1,), (0,)), ((), ())),
                        preferred_element_type=jnp.float32,
                    )
                    m_ref[pl.ds(b * HL + h, 1), :] = m_new

        for b in range(B):
            for h in range(HL):
                ctxT_h = (
                    macc_ref[b, h] / l_ref[pl.ds(b * HL + h, 1), :]
                ).astype(jnp.bfloat16)
                out_ref[b] = out_ref[b] + lax.dot_general(
                    ctxT_h, wo_ref[pl.ds(h * DH, DH), :],
                    (((0,), (0,)), ((), ())),
                    preferred_element_type=jnp.float32,
                )

        pbf_ref[...] = out_ref[...].astype(jnp.bfloat16)

        for d in range(1, N_DEV):
            peer = (my + d) % N_DEV
            rd = pltpu.make_async_remote_copy(
                src_ref=pbf_ref.at[:, pl.ds(peer * QROWS, QROWS), :],
                dst_ref=rs_ref.at[my],
                send_sem=rs_send_sems.at[d - 1],
                recv_sem=rs_recv_sems.at[my],
                device_id=(peer,),
                device_id_type=pl.DeviceIdType.MESH,
            )
            rd.start()
            sends.append(rd)

        for j in range(N_DEV):
            @pl.when(my != j)
            def _(j=j):
                rd = pltpu.make_async_remote_copy(
                    src_ref=rs_ref.at[j], dst_ref=rs_ref.at[j],
                    send_sem=rs_recv_sems.at[j], recv_sem=rs_recv_sems.at[j],
                    device_id=(j,), device_id_type=pl.DeviceIdType.MESH,
                )
                rd.wait_recv()

        red = out_ref[:, pl.ds(my * QROWS, QROWS), :]
        for j in range(N_DEV):
            red = red + rs_ref[j].astype(jnp.float32)
        out_ref[:, pl.ds(my * QROWS, QROWS), :] = red
        agb_ref[...] = red.astype(jnp.bfloat16)

        for d in range(1, N_DEV):
            peer = (my + d) % N_DEV
            rd = pltpu.make_async_remote_copy(
                src_ref=agb_ref,
                dst_ref=ag_ref.at[my],
                send_sem=ag_send_sems.at[d - 1],
                recv_sem=ag_recv_sems.at[my],
                device_id=(peer,),
                device_id_type=pl.DeviceIdType.MESH,
            )
            rd.start()
            sends.append(rd)

        for j in range(N_DEV):
            @pl.when(my != j)
            def _(j=j):
                rd = pltpu.make_async_remote_copy(
                    src_ref=ag_ref.at[j], dst_ref=ag_ref.at[j],
                    send_sem=ag_recv_sems.at[j], recv_sem=ag_recv_sems.at[j],
                    device_id=(j,), device_id_type=pl.DeviceIdType.MESH,
                )
                rd.wait_recv()
                out_ref[:, pl.ds(j * QROWS, QROWS), :] = ag_ref[j].astype(
                    jnp.float32
                )

        for rd in sends:
            rd.wait_send()

    return pl.pallas_call(
        body,
        out_shape=jax.ShapeDtypeStruct((B, SQ, D), jnp.float32),
        in_specs=[pl.BlockSpec(memory_space=pltpu.VMEM)] * 5,
        out_specs=pl.BlockSpec(memory_space=pltpu.VMEM),
        scratch_shapes=[
            pltpu.VMEM((N_DEV, B, HL, DH, SKV_LOC), jnp.bfloat16),
            pltpu.VMEM((N_DEV, B, HL, DH, SKV_LOC), jnp.bfloat16),
            pltpu.VMEM((B, HL * DH, SQ), jnp.bfloat16),
            pltpu.VMEM((B, HL, DH, SQ), jnp.float32),
            pltpu.VMEM((B * HL, SQ), jnp.float32),
            pltpu.VMEM((B * HL, SQ), jnp.float32),
            pltpu.VMEM((N_DEV, B, QROWS, D), jnp.bfloat16),
            pltpu.VMEM((N_DEV, B, QROWS, D), jnp.bfloat16),
            pltpu.VMEM((B, SQ, D), jnp.bfloat16),
            pltpu.VMEM((B, QROWS, D), jnp.bfloat16),
            pltpu.SemaphoreType.DMA((N_DEV - 1,)),
            pltpu.SemaphoreType.DMA((N_DEV - 1,)),
            pltpu.SemaphoreType.DMA((N_DEV,)),
            pltpu.SemaphoreType.DMA((N_DEV,)),
            pltpu.SemaphoreType.DMA((N_DEV - 1,)),
            pltpu.SemaphoreType.DMA((N_DEV,)),
            pltpu.SemaphoreType.DMA((N_DEV - 1,)),
            pltpu.SemaphoreType.DMA((N_DEV,)),
        ],
        compiler_params=pltpu.CompilerParams(collective_id=0),
    )(xt, wqb, kt, vt, wob)


# device time: 103804 ns/iter; 1.3024x vs baseline; 1.0062x over previous
import jax
import jax.numpy as jnp
from jax import lax
from jax.experimental import pallas as pl
from jax.experimental.pallas import tpu as pltpu

N_DEV = 4
B, SQ, D = 2, 512, 768
HL, DH = 8, 64
HD = HL * DH
SKV_LOC = 512
WINDOW, NGLOB = 128, 32
W1 = 128
QROWS = SQ // N_DEV
GL = 640

NEG = -1e9


def kernel(x, Wq, K_ext, V_ext, Wo):
    xt = x.astype(jnp.bfloat16).transpose(0, 2, 1)
    xg = x[:, :NGLOB, :].astype(jnp.bfloat16)
    wqb = Wq.astype(jnp.bfloat16)
    ktv = jnp.stack(
        [K_ext.astype(jnp.bfloat16).transpose(0, 2, 3, 1),
         V_ext.astype(jnp.bfloat16).transpose(0, 2, 3, 1)], axis=0
    )
    wob = Wo.astype(jnp.bfloat16)

    def body(xt_ref, xg_ref, wq_ref, ktv_ref, wo_ref, out_ref,
             kv0_ref, kv1_ref, qt_ref, qg_ref, gout_ref, gin_ref,
             rs_ref, ag_ref, pbf_ref, agb_ref,
             kv0_send_sems, kv0_recv_sem, kv1_send_sems, kv1_recv_sem,
             qg_send_sems, qg_recv_sems, g_send_sems, g_recv_sems,
             rs_send_sems, rs_recv_sems, ag_send_sems, ag_recv_sems):
        my = lax.axis_index("i")

        barrier_sem = pltpu.get_barrier_semaphore()
        for d in range(1, N_DEV):
            pl.semaphore_signal(
                barrier_sem, inc=1,
                device_id=((my + d) % N_DEV,),
                device_id_type=pl.DeviceIdType.MESH,
            )
        pl.semaphore_wait(barrier_sem, N_DEV - 1)

        sends = []

        qg_mine = jnp.stack([
            (lax.dot_general(
                xg_ref[b], wq_ref[...], (((1,), (0,)), ((), ())),
                preferred_element_type=jnp.float32,
            ) * 0.125).astype(jnp.bfloat16)
            for b in range(B)
        ], axis=0)
        qg_ref[pl.ds(my, 1)] = qg_mine[None]
        for d in range(1, N_DEV):
            peer = (my + d) % N_DEV
            rd = pltpu.make_async_remote_copy(
                src_ref=qg_ref.at[my], dst_ref=qg_ref.at[my],
                send_sem=qg_send_sems.at[d - 1], recv_sem=qg_recv_sems.at[my],
                device_id=(peer,), device_id_type=pl.DeviceIdType.MESH,
            )
            rd.start()
            sends.append(rd)

        @pl.when(my == 0)
        def _():
            for d in range(1, N_DEV):
                rd = pltpu.make_async_remote_copy(
                    src_ref=ktv_ref.at[:, :, pl.ds(d * HL, HL)],
                    dst_ref=kv0_ref,
                    send_sem=kv0_send_sems.at[d - 1], recv_sem=kv0_recv_sem.at[0],
                    device_id=(d,), device_id_type=pl.DeviceIdType.MESH,
                )
                rd.start()
            kv0_ref[...] = ktv_ref[:, :, 0:HL]

        @pl.when(my == 1)
        def _():
            for d in range(1, N_DEV):
                peer = (1 + d) % N_DEV
                rd = pltpu.make_async_remote_copy(
                    src_ref=ktv_ref.at[:, :, pl.ds(peer * HL, HL), :, pl.ds(0, W1)],
                    dst_ref=kv1_ref,
                    send_sem=kv1_send_sems.at[d - 1], recv_sem=kv1_recv_sem.at[0],
                    device_id=(peer,), device_id_type=pl.DeviceIdType.MESH,
                )
                rd.start()
            kv1_ref[...] = ktv_ref[:, :, HL:2 * HL, :, 0:W1]

        for b in range(B):
            qt_b = lax.dot_general(
                wq_ref[...], xt_ref[b], (((0,), (0,)), ((), ())),
                preferred_element_type=jnp.float32,
            )
            qt_ref[b] = (qt_b * 0.125).astype(jnp.bfloat16)

        out_ref[...] = jnp.zeros((B, SQ, D), jnp.float32)
        rs_ref[pl.ds(my, 1)] = jnp.zeros((1, B, QROWS, D), jnp.bfloat16)

        for j in range(N_DEV):
            @pl.when(my != j)
            def _(j=j):
                rd = pltpu.make_async_remote_copy(
                    src_ref=qg_ref.at[j], dst_ref=qg_ref.at[j],
                    send_sem=qg_recv_sems.at[j], recv_sem=qg_recv_sems.at[j],
                    device_id=(j,), device_id_type=pl.DeviceIdType.MESH,
                )
                rd.wait_recv()

        for r in range(N_DEV):
            payload = []
            for b in range(B):
                qg_rb = qg_ref[r, b]
                accs, ms, ls = [], [], []
                for h in range(HL):
                    q_h = qg_rb[:, h * DH:(h + 1) * DH]
                    s = lax.dot_general(
                        q_h, ktv_ref[0, b, r * HL + h],
                        (((1,), (0,)), ((), ())),
                        preferred_element_type=jnp.float32,
                    )
                    mh = s.max(axis=1, keepdims=True)
                    p = jnp.exp(s - mh)
                    lh = p.sum(axis=1, keepdims=True)
                    a_h = lax.dot_general(
                        p.astype(jnp.bfloat16), ktv_ref[1, b, r * HL + h],
                        (((1,), (1,)), ((), ())),
                        preferred_element_type=jnp.float32,
                    )
                    accs.append(a_h)
                    ms.append(mh)
                    ls.append(lh)
                payload.append(jnp.concatenate(
                    [jnp.concatenate(accs, axis=1).astype(jnp.bfloat16),
                     jnp.concatenate(ms, axis=1).astype(jnp.bfloat16),
                     jnp.concatenate(ls, axis=1).astype(jnp.bfloat16),
                     jnp.zeros((NGLOB, GL - HD - 2 * HL), jnp.bfloat16)],
                    axis=1,
                ))
            gout_ref[r] = jnp.stack(payload, axis=0)

        for d in range(1, N_DEV):
            peer = (my + d) % N_DEV
            rd = pltpu.make_async_remote_copy(
                src_ref=gout_ref.at[peer], dst_ref=gin_ref.at[my],
                send_sem=g_send_sems.at[d - 1], recv_sem=g_recv_sems.at[my],
                device_id=(peer,), device_id_type=pl.DeviceIdType.MESH,
            )
            rd.start()
            sends.append(rd)
        gin_ref[pl.ds(my, 1)] = gout_ref[my][None]

        @pl.when(my != 1)
        def _():
            rd = pltpu.make_async_remote_copy(
                src_ref=kv1_ref, dst_ref=kv1_ref,
                send_sem=kv1_recv_sem.at[0], recv_sem=kv1_recv_sem.at[0],
                device_id=(1,), device_id_type=pl.DeviceIdType.MESH,
            )
            rd.wait_recv()

        @pl.when(my != 0)
        def _():
            rd = pltpu.make_async_remote_copy(
                src_ref=kv0_ref, dst_ref=kv0_ref,
                send_sem=kv0_recv_sem.at[0], recv_sem=kv0_recv_sem.at[0],
                device_id=(0,), device_id_type=pl.DeviceIdType.MESH,
            )
            rd.wait_recv()

        kvi0 = lax.broadcasted_iota(jnp.int32, (SKV_LOC, SQ), 0)
        qix0 = lax.broadcasted_iota(jnp.int32, (SKV_LOC, SQ), 1)
        mask0 = (jnp.abs(qix0 - kvi0) <= WINDOW) | (kvi0 < NGLOB)
        kvi1 = lax.broadcasted_iota(jnp.int32, (W1, SQ), 0) + SKV_LOC
        qix1 = lax.broadcasted_iota(jnp.int32, (W1, SQ), 1)
        mask1 = jnp.abs(qix1 - kvi1) <= WINDOW
        for b in range(B):
            for h in range(HL):
                qt_h = qt_ref[b, pl.ds(h * DH, DH), :]
                s0 = lax.dot_general(
                    kv0_ref[0, b, h], qt_h, (((0,), (0,)), ((), ())),
                    preferred_element_type=jnp.float32,
                )
                s0 = jnp.where(mask0, s0, NEG)
                m0 = s0.max(axis=0, keepdims=True)
                p0 = jnp.exp(s0 - m0)
                l0 = p0.sum(axis=0, keepdims=True)
                a0 = lax.dot_general(
                    kv0_ref[1, b, h], p0.astype(jnp.bfloat16),
                    (((1,), (0,)), ((), ())),
                    preferred_element_type=jnp.float32,
                )
                s1 = lax.dot_general(
                    kv1_ref[0, b, h], qt_h, (((0,), (0,)), ((), ())),
                    preferred_element_type=jnp.float32,
                )
                s1 = jnp.where(mask1, s1, NEG)
                m1 = s1.max(axis=0, keepdims=True)
                p1 = jnp.exp(s1 - m1)
                l1 = p1.sum(axis=0, keepdims=True)
                a1 = lax.dot_general(
                    kv1_ref[1, b, h], p1.astype(jnp.bfloat16),
                    (((1,), (0,)), ((), ())),
                    preferred_element_type=jnp.float32,
                )
                mm = jnp.maximum(m0, m1)
                w0 = jnp.exp(m0 - mm)
                w1 = jnp.exp(m1 - mm)
                ll = l0 * w0 + l1 * w1
                ctxT = ((a0 * w0 + a1 * w1) / ll).astype(jnp.bfloat16)
                out_ref[b] = out_ref[b] + lax.dot_general(
                    ctxT, wo_ref[pl.ds(h * DH, DH), :],
                    (((0,), (0,)), ((), ())),
                    preferred_element_type=jnp.float32,
                )

        for j in range(N_DEV):
            @pl.when(my != j)
            def _(j=j):
                rd = pltpu.make_async_remote_copy(
                    src_ref=gin_ref.at[j], dst_ref=gin_ref.at[j],
                    send_sem=g_recv_sems.at[j], recv_sem=g_recv_sems.at[j],
                    device_id=(j,), device_id_type=pl.DeviceIdType.MESH,
                )
                rd.wait_recv()

        for b in range(B):
            accs, ms, ls = [], [], []
            for j in range(N_DEV):
                g = gin_ref[j, b]
                accs.append(
                    g[:, :HD].astype(jnp.float32).reshape(NGLOB, HL, DH)
                )
                ms.append(g[:, HD:HD + HL].astype(jnp.float32))
                ls.append(g[:, HD + HL:HD + 2 * HL].astype(jnp.float32))
            mm = jnp.maximum(jnp.maximum(ms[0], ms[1]),
                             jnp.maximum(ms[2], ms[3]))
            num = jnp.zeros((NGLOB, HL, DH), jnp.float32)
            den = jnp.zeros((NGLOB, HL), jnp.float32)
            for j in range(N_DEV):
                w = jnp.exp(ms[j] - mm)
                num = num + accs[j] * w[:, :, None]
                den = den + ls[j] * w
            ctx_g = (num / den[:, :, None]).reshape(NGLOB, HD)
            out_ref[b, pl.ds(0, NGLOB), :] = lax.dot_general(
                ctx_g.astype(jnp.bfloat16), wo_ref[...],
                (((1,), (0,)), ((), ())),
                preferred_element_type=jnp.float32,
            )

        pbf_ref[...] = out_ref[...].astype(jnp.bfloat16)

        for d in range(1, N_DEV):
            peer = (my + d) % N_DEV
            rd = pltpu.make_async_remote_copy(
                src_ref=pbf_ref.at[:, pl.ds(peer * QROWS, QROWS), :],
                dst_ref=rs_ref.at[my],
                send_sem=rs_send_sems.at[d - 1], recv_sem=rs_recv_sems.at[my],
                device_id=(peer,), device_id_type=pl.DeviceIdType.MESH,
            )
            rd.start()
            sends.append(rd)

        for j in range(N_DEV):
            @pl.when(my != j)
            def _(j=j):
                rd = pltpu.make_async_remote_copy(
                    src_ref=rs_ref.at[j], dst_ref=rs_ref.at[j],
                    send_sem=rs_recv_sems.at[j], recv_sem=rs_recv_sems.at[j],
                    device_id=(j,), device_id_type=pl.DeviceIdType.MESH,
                )
                rd.wait_recv()

        red = out_ref[:, pl.ds(my * QROWS, QROWS), :]
        for j in range(N_DEV):
            red = red + rs_ref[j].astype(jnp.float32)
        out_ref[:, pl.ds(my * QROWS, QROWS), :] = red
        agb_ref[...] = red.astype(jnp.bfloat16)

        for d in range(1, N_DEV):
            peer = (my + d) % N_DEV
            rd = pltpu.make_async_remote_copy(
                src_ref=agb_ref, dst_ref=ag_ref.at[my],
                send_sem=ag_send_sems.at[d - 1], recv_sem=ag_recv_sems.at[my],
                device_id=(peer,), device_id_type=pl.DeviceIdType.MESH,
            )
            rd.start()
            sends.append(rd)

        for j in range(N_DEV):
            @pl.when(my != j)
            def _(j=j):
                rd = pltpu.make_async_remote_copy(
                    src_ref=ag_ref.at[j], dst_ref=ag_ref.at[j],
                    send_sem=ag_recv_sems.at[j], recv_sem=ag_recv_sems.at[j],
                    device_id=(j,), device_id_type=pl.DeviceIdType.MESH,
                )
                rd.wait_recv()
                out_ref[:, pl.ds(j * QROWS, QROWS), :] = ag_ref[j].astype(
                    jnp.float32
                )

        for rd in sends:
            rd.wait_send()

        @pl.when(my == 0)
        def _():
            for d in range(1, N_DEV):
                rd = pltpu.make_async_remote_copy(
                    src_ref=ktv_ref.at[:, :, pl.ds(d * HL, HL)],
                    dst_ref=kv0_ref,
                    send_sem=kv0_send_sems.at[d - 1], recv_sem=kv0_recv_sem.at[0],
                    device_id=(d,), device_id_type=pl.DeviceIdType.MESH,
                )
                rd.wait_send()

        @pl.when(my == 1)
        def _():
            for d in range(1, N_DEV):
                peer = (1 + d) % N_DEV
                rd = pltpu.make_async_remote_copy(
                    src_ref=ktv_ref.at[:, :, pl.ds(peer * HL, HL), :, pl.ds(0, W1)],
                    dst_ref=kv1_ref,
                    send_sem=kv1_send_sems.at[d - 1], recv_sem=kv1_recv_sem.at[0],
                    device_id=(peer,), device_id_type=pl.DeviceIdType.MESH,
                )
                rd.wait_send()

    return pl.pallas_call(
        body,
        out_shape=jax.ShapeDtypeStruct((B, SQ, D), jnp.float32),
        in_specs=[pl.BlockSpec(memory_space=pltpu.VMEM)] * 5,
        out_specs=pl.BlockSpec(memory_space=pltpu.VMEM),
        scratch_shapes=[
            pltpu.VMEM((2, B, HL, DH, SKV_LOC), jnp.bfloat16),
            pltpu.VMEM((2, B, HL, DH, W1), jnp.bfloat16),
            pltpu.VMEM((B, HD, SQ), jnp.bfloat16),
            pltpu.VMEM((N_DEV, B, NGLOB, HD), jnp.bfloat16),
            pltpu.VMEM((N_DEV, B, NGLOB, GL), jnp.bfloat16),
            pltpu.VMEM((N_DEV, B, NGLOB, GL), jnp.bfloat16),
            pltpu.VMEM((N_DEV, B, QROWS, D), jnp.bfloat16),
            pltpu.VMEM((N_DEV, B, QROWS, D), jnp.bfloat16),
            pltpu.VMEM((B, SQ, D), jnp.bfloat16),
            pltpu.VMEM((B, QROWS, D), jnp.bfloat16),
            pltpu.SemaphoreType.DMA((N_DEV - 1,)),
            pltpu.SemaphoreType.DMA((1,)),
            pltpu.SemaphoreType.DMA((N_DEV - 1,)),
            pltpu.SemaphoreType.DMA((1,)),
            pltpu.SemaphoreType.DMA((N_DEV - 1,)),
            pltpu.SemaphoreType.DMA((N_DEV,)),
            pltpu.SemaphoreType.DMA((N_DEV - 1,)),
            pltpu.SemaphoreType.DMA((N_DEV,)),
            pltpu.SemaphoreType.DMA((N_DEV - 1,)),
            pltpu.SemaphoreType.DMA((N_DEV,)),
            pltpu.SemaphoreType.DMA((N_DEV - 1,)),
            pltpu.SemaphoreType.DMA((N_DEV,)),
        ],
        compiler_params=pltpu.CompilerParams(collective_id=0),
    )(xt, xg, wqb, ktv, wob)


# device time: 102869 ns/iter; 1.3143x vs baseline; 1.0091x over previous
import jax
import jax.numpy as jnp
from jax import lax
from jax.experimental import pallas as pl
from jax.experimental.pallas import tpu as pltpu

N_DEV = 4
B, SQ, D = 2, 512, 768
HL, DH = 8, 64
HD = HL * DH
SKV_LOC = 512
WINDOW, NGLOB = 128, 32
W1 = 128
QROWS = SQ // N_DEV
GL = 640

NEG = -1e9


def kernel(x, Wq, K_ext, V_ext, Wo):
    xt = x.astype(jnp.bfloat16).transpose(0, 2, 1)
    xg = x[:, :NGLOB, :].astype(jnp.bfloat16)
    wqb = Wq.astype(jnp.bfloat16)
    ktv = jnp.stack(
        [K_ext.astype(jnp.bfloat16).transpose(0, 2, 3, 1),
         V_ext.astype(jnp.bfloat16).transpose(0, 2, 3, 1)], axis=0
    )
    wob = Wo.astype(jnp.bfloat16)

    def body(xt_ref, xg_ref, wq_ref, ktv_ref, wo_ref, out_ref,
             kv0_ref, kv1_ref, qt_ref, qg_ref, gout_ref, gin_ref,
             rs_ref, ag_ref, pbf_ref, agb_ref,
             kv0_send_sems, kv0_recv_sem, kv1_send_sems, kv1_recv_sem,
             qg_send_sems, qg_recv_sems, g_send_sems, g_recv_sems,
             rs_send_sems, rs_recv_sems, ag_send_sems, ag_recv_sems):
        my = lax.axis_index("i")

        barrier_sem = pltpu.get_barrier_semaphore()
        for d in range(1, N_DEV):
            pl.semaphore_signal(
                barrier_sem, inc=1,
                device_id=((my + d) % N_DEV,),
                device_id_type=pl.DeviceIdType.MESH,
            )
        pl.semaphore_wait(barrier_sem, N_DEV - 1)

        sends = []

        qg_mine = jnp.stack([
            (lax.dot_general(
                xg_ref[b], wq_ref[...], (((1,), (0,)), ((), ())),
                preferred_element_type=jnp.float32,
            ) * 0.125).astype(jnp.bfloat16)
            for b in range(B)
        ], axis=0)
        qg_ref[pl.ds(my, 1)] = qg_mine[None]
        for d in range(1, N_DEV):
            peer = (my + d) % N_DEV
            rd = pltpu.make_async_remote_copy(
                src_ref=qg_ref.at[my], dst_ref=qg_ref.at[my],
                send_sem=qg_send_sems.at[d - 1], recv_sem=qg_recv_sems.at[my],
                device_id=(peer,), device_id_type=pl.DeviceIdType.MESH,
            )
            rd.start()
            sends.append(rd)

        @pl.when(my == 0)
        def _():
            for d in range(1, N_DEV):
                rd = pltpu.make_async_remote_copy(
                    src_ref=ktv_ref.at[:, :, pl.ds(d * HL, HL)],
                    dst_ref=kv0_ref,
                    send_sem=kv0_send_sems.at[d - 1], recv_sem=kv0_recv_sem.at[0],
                    device_id=(d,), device_id_type=pl.DeviceIdType.MESH,
                )
                rd.start()
            kv0_ref[...] = ktv_ref[:, :, 0:HL]

        @pl.when(my == 1)
        def _():
            for d in range(1, N_DEV):
                peer = (1 + d) % N_DEV
                rd = pltpu.make_async_remote_copy(
                    src_ref=ktv_ref.at[:, :, pl.ds(peer * HL, HL), :, pl.ds(0, W1)],
                    dst_ref=kv1_ref,
                    send_sem=kv1_send_sems.at[d - 1], recv_sem=kv1_recv_sem.at[0],
                    device_id=(peer,), device_id_type=pl.DeviceIdType.MESH,
                )
                rd.start()
            kv1_ref[...] = ktv_ref[:, :, HL:2 * HL, :, 0:W1]

        for b in range(B):
            qt_b = lax.dot_general(
                wq_ref[...], xt_ref[b], (((0,), (0,)), ((), ())),
                preferred_element_type=jnp.float32,
            )
            qt_ref[b] = (qt_b * 0.125).astype(jnp.bfloat16)

        rs_ref[pl.ds(my, 1)] = jnp.zeros((1, B, QROWS, D), jnp.bfloat16)

        for j in range(N_DEV):
            @pl.when(my != j)
            def _(j=j):
                rd = pltpu.make_async_remote_copy(
                    src_ref=qg_ref.at[j], dst_ref=qg_ref.at[j],
                    send_sem=qg_recv_sems.at[j], recv_sem=qg_recv_sems.at[j],
                    device_id=(j,), device_id_type=pl.DeviceIdType.MESH,
                )
                rd.wait_recv()

        for r in range(N_DEV):
            payload = []
            for b in range(B):
                qg_rb = qg_ref[r, b]
                accs, ms, ls = [], [], []
                for h in range(HL):
                    q_h = qg_rb[:, h * DH:(h + 1) * DH]
                    s = lax.dot_general(
                        q_h, ktv_ref[0, b, r * HL + h],
                        (((1,), (0,)), ((), ())),
                        preferred_element_type=jnp.float32,
                    )
                    mh = s.max(axis=1, keepdims=True)
                    p = jnp.exp(s - mh)
                    lh = p.sum(axis=1, keepdims=True)
                    a_h = lax.dot_general(
                        p.astype(jnp.bfloat16), ktv_ref[1, b, r * HL + h],
                        (((1,), (1,)), ((), ())),
                        preferred_element_type=jnp.float32,
                    )
                    accs.append(a_h)
                    ms.append(mh)
                    ls.append(lh)
                payload.append(jnp.concatenate(
                    [jnp.concatenate(accs, axis=1).astype(jnp.bfloat16),
                     jnp.concatenate(ms, axis=1).astype(jnp.bfloat16),
                     jnp.concatenate(ls, axis=1).astype(jnp.bfloat16),
                     jnp.zeros((NGLOB, GL - HD - 2 * HL), jnp.bfloat16)],
                    axis=1,
                ))
            gout_ref[r] = jnp.stack(payload, axis=0)

        for d in range(1, N_DEV):
            peer = (my + d) % N_DEV
            rd = pltpu.make_async_remote_copy(
                src_ref=gout_ref.at[peer], dst_ref=gin_ref.at[my],
                send_sem=g_send_sems.at[d - 1], recv_sem=g_recv_sems.at[my],
                device_id=(peer,), device_id_type=pl.DeviceIdType.MESH,
            )
            rd.start()
            sends.append(rd)
        gin_ref[pl.ds(my, 1)] = gout_ref[my][None]

        @pl.when(my != 1)
        def _():
            rd = pltpu.make_async_remote_copy(
                src_ref=kv1_ref, dst_ref=kv1_ref,
                send_sem=kv1_recv_sem.at[0], recv_sem=kv1_recv_sem.at[0],
                device_id=(1,), device_id_type=pl.DeviceIdType.MESH,
            )
            rd.wait_recv()

        @pl.when(my != 0)
        def _():
            rd = pltpu.make_async_remote_copy(
                src_ref=kv0_ref, dst_ref=kv0_ref,
                send_sem=kv0_recv_sem.at[0], recv_sem=kv0_recv_sem.at[0],
                device_id=(0,), device_id_type=pl.DeviceIdType.MESH,
            )
            rd.wait_recv()

        kvi0 = lax.broadcasted_iota(jnp.int32, (SKV_LOC, SQ), 0)
        qix0 = lax.broadcasted_iota(jnp.int32, (SKV_LOC, SQ), 1)
        mask0 = (jnp.abs(qix0 - kvi0) <= WINDOW) | (kvi0 < NGLOB)
        kvi1 = lax.broadcasted_iota(jnp.int32, (W1, SQ), 0) + SKV_LOC
        qix1 = lax.broadcasted_iota(jnp.int32, (W1, SQ), 1)
        mask1 = jnp.abs(qix1 - kvi1) <= WINDOW
        for b in range(B):
            out_b = None
            for h in range(HL):
                qt_h = qt_ref[b, pl.ds(h * DH, DH), :]
                s0 = lax.dot_general(
                    kv0_ref[0, b, h], qt_h, (((0,), (0,)), ((), ())),
                    preferred_element_type=jnp.float32,
                )
                s0 = jnp.where(mask0, s0, NEG)
                m0 = s0.max(axis=0, keepdims=True)
                p0 = jnp.exp(s0 - m0)
                l0 = p0.sum(axis=0, keepdims=True)
                a0 = lax.dot_general(
                    kv0_ref[1, b, h], p0.astype(jnp.bfloat16),
                    (((1,), (0,)), ((), ())),
                    preferred_element_type=jnp.float32,
                )
                s1 = lax.dot_general(
                    kv1_ref[0, b, h], qt_h, (((0,), (0,)), ((), ())),
                    preferred_element_type=jnp.float32,
                )
                s1 = jnp.where(mask1, s1, NEG)
                m1 = s1.max(axis=0, keepdims=True)
                p1 = jnp.exp(s1 - m1)
                l1 = p1.sum(axis=0, keepdims=True)
                a1 = lax.dot_general(
                    kv1_ref[1, b, h], p1.astype(jnp.bfloat16),
                    (((1,), (0,)), ((), ())),
                    preferred_element_type=jnp.float32,
                )
                mm = jnp.maximum(m0, m1)
                w0 = jnp.exp(m0 - mm)
                w1 = jnp.exp(m1 - mm)
                ll = l0 * w0 + l1 * w1
                ctxT = ((a0 * w0 + a1 * w1) / ll).astype(jnp.bfloat16)
                contrib = lax.dot_general(
                    ctxT, wo_ref[pl.ds(h * DH, DH), :],
                    (((0,), (0,)), ((), ())),
                    preferred_element_type=jnp.float32,
                )
                out_b = contrib if out_b is None else out_b + contrib
            out_ref[b] = out_b

        for j in range(N_DEV):
            @pl.when(my != j)
            def _(j=j):
                rd = pltpu.make_async_remote_copy(
                    src_ref=gin_ref.at[j], dst_ref=gin_ref.at[j],
                    send_sem=g_recv_sems.at[j], recv_sem=g_recv_sems.at[j],
                    device_id=(j,), device_id_type=pl.DeviceIdType.MESH,
                )
                rd.wait_recv()

        for b in range(B):
            accs, ms, ls = [], [], []
            for j in range(N_DEV):
                g = gin_ref[j, b]
                accs.append(
                    g[:, :HD].astype(jnp.float32).reshape(NGLOB, HL, DH)
                )
                ms.append(g[:, HD:HD + HL].astype(jnp.float32))
                ls.append(g[:, HD + HL:HD + 2 * HL].astype(jnp.float32))
            mm = jnp.maximum(jnp.maximum(ms[0], ms[1]),
                             jnp.maximum(ms[2], ms[3]))
            num = jnp.zeros((NGLOB, HL, DH), jnp.float32)
            den = jnp.zeros((NGLOB, HL), jnp.float32)
            for j in range(N_DEV):
                w = jnp.exp(ms[j] - mm)
                num = num + accs[j] * w[:, :, None]
                den = den + ls[j] * w
            ctx_g = (num / den[:, :, None]).reshape(NGLOB, HD)
            out_ref[b, pl.ds(0, NGLOB), :] = lax.dot_general(
                ctx_g.astype(jnp.bfloat16), wo_ref[...],
                (((1,), (0,)), ((), ())),
                preferred_element_type=jnp.float32,
            )

        pbf_ref[...] = out_ref[...].astype(jnp.bfloat16)

        for d in range(1, N_DEV):
            peer = (my + d) % N_DEV
            rd = pltpu.make_async_remote_copy(
                src_ref=pbf_ref.at[:, pl.ds(peer * QROWS, QROWS), :],
                dst_ref=rs_ref.at[my],
                send_sem=rs_send_sems.at[d - 1], recv_sem=rs_recv_sems.at[my],
                device_id=(peer,), device_id_type=pl.DeviceIdType.MESH,
            )
            rd.start()
            sends.append(rd)

        for j in range(N_DEV):
            @pl.when(my != j)
            def _(j=j):
                rd = pltpu.make_async_remote_copy(
                    src_ref=rs_ref.at[j], dst_ref=rs_ref.at[j],
                    send_sem=rs_recv_sems.at[j], recv_sem=rs_recv_sems.at[j],
                    device_id=(j,), device_id_type=pl.DeviceIdType.MESH,
                )
                rd.wait_recv()

        red = out_ref[:, pl.ds(my * QROWS, QROWS), :]
        for j in range(N_DEV):
            red = red + rs_ref[j].astype(jnp.float32)
        out_ref[:, pl.ds(my * QROWS, QROWS), :] = red
        agb_ref[...] = red.astype(jnp.bfloat16)

        for d in range(1, N_DEV):
            peer = (my + d) % N_DEV
            rd = pltpu.make_async_remote_copy(
                src_ref=agb_ref, dst_ref=ag_ref.at[my],
                send_sem=ag_send_sems.at[d - 1], recv_sem=ag_recv_sems.at[my],
                device_id=(peer,), device_id_type=pl.DeviceIdType.MESH,
            )
            rd.start()
            sends.append(rd)

        for j in range(N_DEV):
            @pl.when(my != j)
            def _(j=j):
                rd = pltpu.make_async_remote_copy(
                    src_ref=ag_ref.at[j], dst_ref=ag_ref.at[j],
                    send_sem=ag_recv_sems.at[j], recv_sem=ag_recv_sems.at[j],
                    device_id=(j,), device_id_type=pl.DeviceIdType.MESH,
                )
                rd.wait_recv()
                out_ref[:, pl.ds(j * QROWS, QROWS), :] = ag_ref[j].astype(
                    jnp.float32
                )

        for rd in sends:
            rd.wait_send()

        @pl.when(my == 0)
        def _():
            for d in range(1, N_DEV):
                rd = pltpu.make_async_remote_copy(
                    src_ref=ktv_ref.at[:, :, pl.ds(d * HL, HL)],
                    dst_ref=kv0_ref,
                    send_sem=kv0_send_sems.at[d - 1], recv_sem=kv0_recv_sem.at[0],
                    device_id=(d,), device_id_type=pl.DeviceIdType.MESH,
                )
                rd.wait_send()

        @pl.when(my == 1)
        def _():
            for d in range(1, N_DEV):
                peer = (1 + d) % N_DEV
                rd = pltpu.make_async_remote_copy(
                    src_ref=ktv_ref.at[:, :, pl.ds(peer * HL, HL), :, pl.ds(0, W1)],
                    dst_ref=kv1_ref,
                    send_sem=kv1_send_sems.at[d - 1], recv_sem=kv1_recv_sem.at[0],
                    device_id=(peer,), device_id_type=pl.DeviceIdType.MESH,
                )
                rd.wait_send()

    return pl.pallas_call(
        body,
        out_shape=jax.ShapeDtypeStruct((B, SQ, D), jnp.float32),
        in_specs=[pl.BlockSpec(memory_space=pltpu.VMEM)] * 5,
        out_specs=pl.BlockSpec(memory_space=pltpu.VMEM),
        scratch_shapes=[
            pltpu.VMEM((2, B, HL, DH, SKV_LOC), jnp.bfloat16),
            pltpu.VMEM((2, B, HL, DH, W1), jnp.bfloat16),
            pltpu.VMEM((B, HD, SQ), jnp.bfloat16),
            pltpu.VMEM((N_DEV, B, NGLOB, HD), jnp.bfloat16),
            pltpu.VMEM((N_DEV, B, NGLOB, GL), jnp.bfloat16),
            pltpu.VMEM((N_DEV, B, NGLOB, GL), jnp.bfloat16),
            pltpu.VMEM((N_DEV, B, QROWS, D), jnp.bfloat16),
            pltpu.VMEM((N_DEV, B, QROWS, D), jnp.bfloat16),
            pltpu.VMEM((B, SQ, D), jnp.bfloat16),
            pltpu.VMEM((B, QROWS, D), jnp.bfloat16),
            pltpu.SemaphoreType.DMA((N_DEV - 1,)),
            pltpu.SemaphoreType.DMA((1,)),
            pltpu.SemaphoreType.DMA((N_DEV - 1,)),
            pltpu.SemaphoreType.DMA((1,)),
            pltpu.SemaphoreType.DMA((N_DEV - 1,)),
            pltpu.SemaphoreType.DMA((N_DEV,)),
            pltpu.SemaphoreType.DMA((N_DEV - 1,)),
            pltpu.SemaphoreType.DMA((N_DEV,)),
            pltpu.SemaphoreType.DMA((N_DEV - 1,)),
            pltpu.SemaphoreType.DMA((N_DEV,)),
            pltpu.SemaphoreType.DMA((N_DEV - 1,)),
            pltpu.SemaphoreType.DMA((N_DEV,)),
        ],
        compiler_params=pltpu.CompilerParams(collective_id=0),
    )(xt, xg, wqb, ktv, wob)
